# baseline (speedup 1.0000x reference)
import jax
import jax.numpy as jnp
from jax.experimental import pallas as pl
from jax.experimental.pallas import tpu as pltpu

HIDDEN = 2048
NUM_EXPERTS = 16
TOKENS = 16384
BLOCK = 1024
NBLOCKS = TOKENS // BLOCK


def _top2(logits):
    lane = jax.lax.broadcasted_iota(jnp.int32, logits.shape, 1)
    m1 = jnp.max(logits, axis=1, keepdims=True)
    i1 = jnp.min(jnp.where(logits == m1, lane, NUM_EXPERTS), axis=1,
                 keepdims=True)
    masked = jnp.where(lane == i1, -jnp.inf, logits)
    m2 = jnp.max(masked, axis=1, keepdims=True)
    i2 = jnp.min(jnp.where(masked == m2, lane, NUM_EXPERTS), axis=1,
                 keepdims=True)
    r = jnp.exp(m2 - m1)
    w1 = 1.0 / (1.0 + r)
    return (jnp.concatenate([w1, 1.0 - w1], axis=1),
            jnp.concatenate([i1, i2], axis=1))


def _router_kernel(x_hbm, wt_ref, w_out_ref, i_out_ref, buf, sems):
    def start_copy(i):
        slot = jax.lax.rem(i, 2)
        pltpu.make_async_copy(
            x_hbm.at[pl.ds(i * BLOCK, BLOCK), :],
            buf.at[slot],
            sems.at[slot],
        ).start()

    def wait_copy(i):
        slot = jax.lax.rem(i, 2)
        pltpu.make_async_copy(
            x_hbm.at[pl.ds(i * BLOCK, BLOCK), :],
            buf.at[slot],
            sems.at[slot],
        ).wait()

    start_copy(0)

    def body(i, _):
        @pl.when(i + 1 < NBLOCKS)
        def _():
            start_copy(i + 1)

        wait_copy(i)
        slot = jax.lax.rem(i, 2)
        logits = jnp.dot(buf[slot], wt_ref[...],
                         preferred_element_type=jnp.float32)
        w_out, i_out = _top2(logits)
        w_out_ref[pl.ds(i * BLOCK, BLOCK), :] = w_out
        i_out_ref[pl.ds(i * BLOCK, BLOCK), :] = i_out
        return 0

    jax.lax.fori_loop(0, NBLOCKS, body, 0)


@jax.jit
def kernel(x, W):
    w_out, i_out = pl.pallas_call(
        _router_kernel,
        in_specs=[
            pl.BlockSpec(memory_space=pltpu.MemorySpace.HBM),
            pl.BlockSpec(memory_space=pltpu.MemorySpace.VMEM),
        ],
        out_specs=[
            pl.BlockSpec(memory_space=pltpu.MemorySpace.VMEM),
            pl.BlockSpec(memory_space=pltpu.MemorySpace.VMEM),
        ],
        out_shape=[
            jax.ShapeDtypeStruct((TOKENS, 2), jnp.float32),
            jax.ShapeDtypeStruct((TOKENS, 2), jnp.int32),
        ],
        scratch_shapes=[
            pltpu.VMEM((2, BLOCK, HIDDEN), jnp.float32),
            pltpu.SemaphoreType.DMA((2,)),
        ],
    )(x, W.T)
    return (w_out, i_out)


# speedup vs baseline: 1.0967x; 1.0967x over previous
import jax
import jax.numpy as jnp
from jax.experimental import pallas as pl
from jax.experimental.pallas import tpu as pltpu

HIDDEN = 2048
NUM_EXPERTS = 16
TOKENS = 16384
BLOCK = 1024


def _router_kernel(x_ref, wt_ref, w_out_ref, i_out_ref):
    logits = jnp.dot(x_ref[...], wt_ref[...],
                     preferred_element_type=jnp.float32)
    lt = logits.T
    row = jax.lax.broadcasted_iota(jnp.int32, lt.shape, 0)
    m1 = jnp.max(lt, axis=0, keepdims=True)
    i1 = jnp.min(jnp.where(lt == m1, row, NUM_EXPERTS), axis=0,
                 keepdims=True)
    masked = jnp.where(row == i1, -jnp.inf, lt)
    m2 = jnp.max(masked, axis=0, keepdims=True)
    i2 = jnp.min(jnp.where(masked == m2, row, NUM_EXPERTS), axis=0,
                 keepdims=True)
    r = jnp.exp(m2 - m1)
    w1 = 1.0 / (1.0 + r)
    wt2 = jnp.concatenate([w1, 1.0 - w1], axis=0)
    it2 = jnp.concatenate([i1, i2], axis=0)
    w_out_ref[...] = wt2.T
    i_out_ref[...] = it2.T


@jax.jit
def kernel(x, W):
    grid = (TOKENS // BLOCK,)
    w_out, i_out = pl.pallas_call(
        _router_kernel,
        grid=grid,
        in_specs=[
            pl.BlockSpec((BLOCK, HIDDEN), lambda i: (i, 0)),
            pl.BlockSpec((HIDDEN, NUM_EXPERTS), lambda i: (0, 0)),
        ],
        out_specs=[
            pl.BlockSpec((BLOCK, 2), lambda i: (i, 0)),
            pl.BlockSpec((BLOCK, 2), lambda i: (i, 0)),
        ],
        out_shape=[
            jax.ShapeDtypeStruct((TOKENS, 2), jnp.float32),
            jax.ShapeDtypeStruct((TOKENS, 2), jnp.int32),
        ],
        compiler_params=pltpu.CompilerParams(
            dimension_semantics=("arbitrary",),
        ),
    )(x, W.T)
    return (w_out, i_out)
